# initial kernel scaffold (unmeasured)
import jax
import jax.numpy as jnp
from jax import lax
from jax.experimental import pallas as pl
from jax.experimental.pallas import tpu as pltpu


def kernel(
    t,
):
    def body(*refs):
        pass

    out_shape = jax.ShapeDtypeStruct(..., jnp.float32)
    return pl.pallas_call(body, out_shape=out_shape)(...)


# baseline (device time: 79908 ns/iter reference)
import jax
import jax.numpy as jnp
from jax import lax
from jax.experimental import pallas as pl
from jax.experimental.pallas import tpu as pltpu

N_DEV = 4


def kernel(t):
    m, n = t.shape

    def body(x_ref, out_ref, comm_ref, send_sems, recv_sems):
        my_pos = lax.axis_index("i")
        left = (my_pos - 1) % N_DEV
        right = (my_pos + 1) % N_DEV

        barrier_sem = pltpu.get_barrier_semaphore()
        for nbr in [left, right]:
            pl.semaphore_signal(
                barrier_sem, inc=1,
                device_id=(nbr,), device_id_type=pl.DeviceIdType.MESH,
            )
        pl.semaphore_wait(barrier_sem, 2)

        comm_ref[0, :, :] = x_ref[:, :]
        out_ref[:, :] = x_ref[:, :]

        for h in range(N_DEV - 1):
            rdma = pltpu.make_async_remote_copy(
                src_ref=comm_ref.at[h],
                dst_ref=comm_ref.at[h + 1],
                send_sem=send_sems.at[h],
                recv_sem=recv_sems.at[h],
                device_id=(right,),
                device_id_type=pl.DeviceIdType.MESH,
            )
            rdma.start()
            rdma.wait()
            out_ref[:, :] += comm_ref[h + 1, :, :]

        s = out_ref[:, :]
        r = jnp.maximum(s, 0.0)
        out_ref[:, :] = jnp.tanh(s) * s * s + r * r * r

    return pl.pallas_call(
        body,
        out_shape=jax.ShapeDtypeStruct((m, n), jnp.float32),
        in_specs=[pl.BlockSpec(memory_space=pltpu.VMEM)],
        out_specs=pl.BlockSpec(memory_space=pltpu.VMEM),
        scratch_shapes=[
            pltpu.VMEM((N_DEV, m, n), jnp.float32),
            pltpu.SemaphoreType.DMA((N_DEV - 1,)),
            pltpu.SemaphoreType.DMA((N_DEV - 1,)),
        ],
        compiler_params=pltpu.CompilerParams(collective_id=0),
    )(t)


# device time: 29225 ns/iter; 2.7342x vs baseline; 2.7342x over previous
import jax
import jax.numpy as jnp
from jax import lax
from jax.experimental import pallas as pl
from jax.experimental.pallas import tpu as pltpu

N_DEV = 4


def _f(s):
    r = jnp.maximum(s, 0.0)
    return jnp.tanh(s) * s * s + r * r * r


def kernel(t):
    m, n = t.shape
    h = m // 4
    q = m // 8

    def body(x_ref, out_ref, rA1, rB1, rA2, rB2, halfA, halfB,
             send_sems, recv_sems):
        my = lax.axis_index("i")
        hx = my // 2
        hy = jnp.bitwise_xor(my, my // 2) % 2
        xp = 3 - my
        yp = my ^ 1

        barrier_sem = pltpu.get_barrier_semaphore()
        for nbr in [xp, yp]:
            pl.semaphore_signal(
                barrier_sem, inc=1,
                device_id=(nbr,), device_id_type=pl.DeviceIdType.MESH,
            )
        pl.semaphore_wait(barrier_sem, 2)

        def exchange(stage, src_a, dst_a, dev_a, src_b, dst_b, dev_b):
            rdma_a = pltpu.make_async_remote_copy(
                src_ref=src_a, dst_ref=dst_a,
                send_sem=send_sems.at[stage, 0], recv_sem=recv_sems.at[stage, 0],
                device_id=(dev_a,), device_id_type=pl.DeviceIdType.MESH,
            )
            rdma_b = pltpu.make_async_remote_copy(
                src_ref=src_b, dst_ref=dst_b,
                send_sem=send_sems.at[stage, 1], recv_sem=recv_sems.at[stage, 1],
                device_id=(dev_b,), device_id_type=pl.DeviceIdType.MESH,
            )
            rdma_a.start()
            rdma_b.start()
            rdma_a.wait()
            rdma_b.wait()

        exchange(
            0,
            x_ref.at[pl.ds((1 - hx) * h, h)], rA1, xp,
            x_ref.at[pl.ds(m // 2 + (1 - hy) * h, h)], rB1, yp,
        )
        halfA[:, :] = x_ref[pl.ds(hx * h, h), :] + rA1[:, :]
        halfB[:, :] = x_ref[pl.ds(m // 2 + hy * h, h), :] + rB1[:, :]

        exchange(
            1,
            halfA.at[pl.ds((1 - hy) * q, q)], rA2, yp,
            halfB.at[pl.ds((1 - hx) * q, q)], rB2, xp,
        )
        qA = 2 * hx + hy
        qB = 2 * hy + hx

        out_ref[pl.ds(qA * q, q), :] = _f(
            halfA[pl.ds(hy * q, q), :] + rA2[:, :])
        out_ref[pl.ds(m // 2 + qB * q, q), :] = _f(
            halfB[pl.ds(hx * q, q), :] + rB2[:, :])

        exchange(
            2,
            out_ref.at[pl.ds(qA * q, q)], out_ref.at[pl.ds(qA * q, q)], yp,
            out_ref.at[pl.ds(m // 2 + qB * q, q)],
            out_ref.at[pl.ds(m // 2 + qB * q, q)], xp,
        )

        exchange(
            3,
            out_ref.at[pl.ds(hx * h, h)], out_ref.at[pl.ds(hx * h, h)], xp,
            out_ref.at[pl.ds(m // 2 + hy * h, h)],
            out_ref.at[pl.ds(m // 2 + hy * h, h)], yp,
        )

    return pl.pallas_call(
        body,
        out_shape=jax.ShapeDtypeStruct((m, n), jnp.float32),
        in_specs=[pl.BlockSpec(memory_space=pltpu.VMEM)],
        out_specs=pl.BlockSpec(memory_space=pltpu.VMEM),
        scratch_shapes=[
            pltpu.VMEM((h, n), jnp.float32),
            pltpu.VMEM((h, n), jnp.float32),
            pltpu.VMEM((q, n), jnp.float32),
            pltpu.VMEM((q, n), jnp.float32),
            pltpu.VMEM((h, n), jnp.float32),
            pltpu.VMEM((h, n), jnp.float32),
            pltpu.SemaphoreType.DMA((4, 2)),
            pltpu.SemaphoreType.DMA((4, 2)),
        ],
        compiler_params=pltpu.CompilerParams(collective_id=0),
    )(t)


# device time: 28121 ns/iter; 2.8416x vs baseline; 1.0393x over previous
import jax
import jax.numpy as jnp
from jax import lax
from jax.experimental import pallas as pl
from jax.experimental.pallas import tpu as pltpu

N_DEV = 4


def kernel(t):
    m, n = t.shape
    h = m // 4
    q = m // 8
    b_off = m // 2

    def body(x_ref, out_ref, rA1, rB1, rA2, rB2, fwdA, fwdB,
             send_sems, recv_sems):
        my = lax.axis_index("i")
        hx = my // 2
        hy = jnp.bitwise_xor(my, my // 2) % 2
        xp = 3 - my
        yp = my ^ 1
        qA = 2 * hx + hy
        qB = 2 * hy + hx

        barrier_sem = pltpu.get_barrier_semaphore()
        for nbr in [xp, yp]:
            pl.semaphore_signal(
                barrier_sem, inc=1,
                device_id=(nbr,), device_id_type=pl.DeviceIdType.MESH,
            )
        pl.semaphore_wait(barrier_sem, 2)

        def rdma(src, dst, sem, dev):
            return pltpu.make_async_remote_copy(
                src_ref=src, dst_ref=dst,
                send_sem=send_sems.at[sem], recv_sem=recv_sems.at[sem],
                device_id=(dev,), device_id_type=pl.DeviceIdType.MESH,
            )

        a0a = rdma(x_ref.at[pl.ds((1 - hx) * h + (1 - hy) * q, q)],
                   rA1.at[pl.ds((1 - hy) * q, q)], 0, xp)
        b0a = rdma(x_ref.at[pl.ds(b_off + (1 - hy) * h + (1 - hx) * q, q)],
                   rB1.at[pl.ds((1 - hx) * q, q)], 1, yp)
        a0b = rdma(x_ref.at[pl.ds((1 - hx) * h + hy * q, q)],
                   rA1.at[pl.ds(hy * q, q)], 2, xp)
        b0b = rdma(x_ref.at[pl.ds(b_off + (1 - hy) * h + hx * q, q)],
                   rB1.at[pl.ds(hx * q, q)], 3, yp)
        a0a.start()
        b0a.start()
        a0b.start()
        b0b.start()

        a0a.wait_recv()
        fwdA[:, :] = (x_ref[pl.ds(hx * h + (1 - hy) * q, q), :]
                      + rA1[pl.ds((1 - hy) * q, q), :])
        a1 = rdma(fwdA, rA2, 4, yp)
        a1.start()

        b0a.wait_recv()
        fwdB[:, :] = (x_ref[pl.ds(b_off + hy * h + (1 - hx) * q, q), :]
                      + rB1[pl.ds((1 - hx) * q, q), :])
        b1 = rdma(fwdB, rB2, 5, xp)
        b1.start()

        a0b.wait_recv()
        a1.wait_recv()
        sA = (x_ref[pl.ds(hx * h + hy * q, q), :]
              + rA1[pl.ds(hy * q, q), :] + rA2[:, :])
        rA = jnp.maximum(sA, 0.0)
        out_ref[pl.ds(qA * q, q), :] = jnp.tanh(sA) * sA * sA + rA * rA * rA
        a2 = rdma(out_ref.at[pl.ds(qA * q, q)],
                  out_ref.at[pl.ds(qA * q, q)], 6, yp)
        a3a = rdma(out_ref.at[pl.ds(qA * q, q)],
                   out_ref.at[pl.ds(qA * q, q)], 7, xp)
        a2.start()
        a3a.start()

        b0b.wait_recv()
        b1.wait_recv()
        sB = (x_ref[pl.ds(b_off + hy * h + hx * q, q), :]
              + rB1[pl.ds(hx * q, q), :] + rB2[:, :])
        rB = jnp.maximum(sB, 0.0)
        out_ref[pl.ds(b_off + qB * q, q), :] = (
            jnp.tanh(sB) * sB * sB + rB * rB * rB)
        b2 = rdma(out_ref.at[pl.ds(b_off + qB * q, q)],
                  out_ref.at[pl.ds(b_off + qB * q, q)], 8, xp)
        b3a = rdma(out_ref.at[pl.ds(b_off + qB * q, q)],
                   out_ref.at[pl.ds(b_off + qB * q, q)], 9, yp)
        b2.start()
        b3a.start()

        qA2 = 2 * hx + (1 - hy)
        qB2 = 2 * hy + (1 - hx)
        a2.wait_recv()
        a3b = rdma(out_ref.at[pl.ds(qA2 * q, q)],
                   out_ref.at[pl.ds(qA2 * q, q)], 10, xp)
        a3b.start()
        b2.wait_recv()
        b3b = rdma(out_ref.at[pl.ds(b_off + qB2 * q, q)],
                   out_ref.at[pl.ds(b_off + qB2 * q, q)], 11, yp)
        b3b.start()

        a3a.wait_recv()
        b3a.wait_recv()
        a3b.wait_recv()
        b3b.wait_recv()

        for d in [a0a, b0a, a0b, b0b, a1, b1, a2, a3a, b2, b3a, a3b, b3b]:
            d.wait_send()

    return pl.pallas_call(
        body,
        out_shape=jax.ShapeDtypeStruct((m, n), jnp.float32),
        in_specs=[pl.BlockSpec(memory_space=pltpu.VMEM)],
        out_specs=pl.BlockSpec(memory_space=pltpu.VMEM),
        scratch_shapes=[
            pltpu.VMEM((h, n), jnp.float32),
            pltpu.VMEM((h, n), jnp.float32),
            pltpu.VMEM((q, n), jnp.float32),
            pltpu.VMEM((q, n), jnp.float32),
            pltpu.VMEM((q, n), jnp.float32),
            pltpu.VMEM((q, n), jnp.float32),
            pltpu.SemaphoreType.DMA((12,)),
            pltpu.SemaphoreType.DMA((12,)),
        ],
        compiler_params=pltpu.CompilerParams(collective_id=0),
    )(t)


# device time: 19933 ns/iter; 4.0088x vs baseline; 1.4108x over previous
import jax
import jax.numpy as jnp
from jax import lax
from jax.experimental import pallas as pl
from jax.experimental.pallas import tpu as pltpu

N_DEV = 4


def kernel(t):
    m, n = t.shape
    h = m // 4
    q = m // 8
    b_off = m // 2
    bf16 = jnp.bfloat16
    f32 = jnp.float32

    def body(x_ref, out_ref, sA0, sB0, rA1, rB1, rA2, rB2, fwdA, fwdB,
             obuf, send_sems, recv_sems):
        my = lax.axis_index("i")
        hx = my // 2
        hy = jnp.bitwise_xor(my, my // 2) % 2
        xp = 3 - my
        yp = my ^ 1
        qA = 2 * hx + hy
        qB = 2 * hy + hx

        barrier_sem = pltpu.get_barrier_semaphore()
        for nbr in [xp, yp]:
            pl.semaphore_signal(
                barrier_sem, inc=1,
                device_id=(nbr,), device_id_type=pl.DeviceIdType.MESH,
            )
        pl.semaphore_wait(barrier_sem, 2)

        def rdma(src, dst, sem, dev):
            return pltpu.make_async_remote_copy(
                src_ref=src, dst_ref=dst,
                send_sem=send_sems.at[sem], recv_sem=recv_sems.at[sem],
                device_id=(dev,), device_id_type=pl.DeviceIdType.MESH,
            )

        sA0[pl.ds((1 - hy) * q, q), :] = x_ref[
            pl.ds((1 - hx) * h + (1 - hy) * q, q), :].astype(bf16)
        sB0[pl.ds((1 - hx) * q, q), :] = x_ref[
            pl.ds(b_off + (1 - hy) * h + (1 - hx) * q, q), :].astype(bf16)

        a0a = rdma(sA0.at[pl.ds((1 - hy) * q, q)],
                   rA1.at[pl.ds((1 - hy) * q, q)], 0, xp)
        b0a = rdma(sB0.at[pl.ds((1 - hx) * q, q)],
                   rB1.at[pl.ds((1 - hx) * q, q)], 1, yp)
        a0a.start()
        b0a.start()

        sA0[pl.ds(hy * q, q), :] = x_ref[
            pl.ds((1 - hx) * h + hy * q, q), :].astype(bf16)
        sB0[pl.ds(hx * q, q), :] = x_ref[
            pl.ds(b_off + (1 - hy) * h + hx * q, q), :].astype(bf16)
        a0b = rdma(sA0.at[pl.ds(hy * q, q)],
                   rA1.at[pl.ds(hy * q, q)], 2, xp)
        b0b = rdma(sB0.at[pl.ds(hx * q, q)],
                   rB1.at[pl.ds(hx * q, q)], 3, yp)
        a0b.start()
        b0b.start()

        a0a.wait_recv()
        fwdA[:, :] = (x_ref[pl.ds(hx * h + (1 - hy) * q, q), :]
                      + rA1[pl.ds((1 - hy) * q, q), :].astype(f32)
                      ).astype(bf16)
        a1 = rdma(fwdA, rA2, 4, yp)
        a1.start()

        b0a.wait_recv()
        fwdB[:, :] = (x_ref[pl.ds(b_off + hy * h + (1 - hx) * q, q), :]
                      + rB1[pl.ds((1 - hx) * q, q), :].astype(f32)
                      ).astype(bf16)
        b1 = rdma(fwdB, rB2, 5, xp)
        b1.start()

        a0b.wait_recv()
        a1.wait_recv()
        sA = (x_ref[pl.ds(hx * h + hy * q, q), :]
              + rA1[pl.ds(hy * q, q), :].astype(f32)
              + rA2[:, :].astype(f32))
        rA = jnp.maximum(sA, 0.0)
        obuf[pl.ds(qA * q, q), :] = (
            jnp.tanh(sA) * sA * sA + rA * rA * rA).astype(bf16)
        a2 = rdma(obuf.at[pl.ds(qA * q, q)],
                  obuf.at[pl.ds(qA * q, q)], 6, yp)
        a3a = rdma(obuf.at[pl.ds(qA * q, q)],
                   obuf.at[pl.ds(qA * q, q)], 7, xp)
        a2.start()
        a3a.start()

        b0b.wait_recv()
        b1.wait_recv()
        sB = (x_ref[pl.ds(b_off + hy * h + hx * q, q), :]
              + rB1[pl.ds(hx * q, q), :].astype(f32)
              + rB2[:, :].astype(f32))
        rB = jnp.maximum(sB, 0.0)
        obuf[pl.ds(b_off + qB * q, q), :] = (
            jnp.tanh(sB) * sB * sB + rB * rB * rB).astype(bf16)
        b2 = rdma(obuf.at[pl.ds(b_off + qB * q, q)],
                  obuf.at[pl.ds(b_off + qB * q, q)], 8, xp)
        b3a = rdma(obuf.at[pl.ds(b_off + qB * q, q)],
                   obuf.at[pl.ds(b_off + qB * q, q)], 9, yp)
        b2.start()
        b3a.start()

        qA2 = 2 * hx + (1 - hy)
        qB2 = 2 * hy + (1 - hx)
        a2.wait_recv()
        a3b = rdma(obuf.at[pl.ds(qA2 * q, q)],
                   obuf.at[pl.ds(qA2 * q, q)], 10, xp)
        a3b.start()
        b2.wait_recv()
        b3b = rdma(obuf.at[pl.ds(b_off + qB2 * q, q)],
                   obuf.at[pl.ds(b_off + qB2 * q, q)], 11, yp)
        b3b.start()

        a3a.wait_recv()
        b3a.wait_recv()
        a3b.wait_recv()
        b3b.wait_recv()

        out_ref[:, :] = obuf[:, :].astype(f32)

        for d in [a0a, b0a, a0b, b0b, a1, b1, a2, a3a, b2, b3a, a3b, b3b]:
            d.wait_send()

    return pl.pallas_call(
        body,
        out_shape=jax.ShapeDtypeStruct((m, n), jnp.float32),
        in_specs=[pl.BlockSpec(memory_space=pltpu.VMEM)],
        out_specs=pl.BlockSpec(memory_space=pltpu.VMEM),
        scratch_shapes=[
            pltpu.VMEM((h, n), bf16),
            pltpu.VMEM((h, n), bf16),
            pltpu.VMEM((h, n), bf16),
            pltpu.VMEM((h, n), bf16),
            pltpu.VMEM((q, n), bf16),
            pltpu.VMEM((q, n), bf16),
            pltpu.VMEM((q, n), bf16),
            pltpu.VMEM((q, n), bf16),
            pltpu.VMEM((m, n), bf16),
            pltpu.SemaphoreType.DMA((12,)),
            pltpu.SemaphoreType.DMA((12,)),
        ],
        compiler_params=pltpu.CompilerParams(collective_id=0),
    )(t)


# device time: 19789 ns/iter; 4.0380x vs baseline; 1.0073x over previous
import jax
import jax.numpy as jnp
from jax import lax
from jax.experimental import pallas as pl
from jax.experimental.pallas import tpu as pltpu

N_DEV = 4


def kernel(t):
    m, n = t.shape
    h = m // 4
    q = m // 8
    b_off = m // 2
    bf16 = jnp.bfloat16
    f32 = jnp.float32

    def body(x_ref, out_ref, sA0, sB0, rA1, rB1, rA2, rB2, fwdA, fwdB,
             obuf, send_sems, recv_sems):
        my = lax.axis_index("i")
        hx = my // 2
        hy = jnp.bitwise_xor(my, my // 2) % 2
        xp = 3 - my
        yp = my ^ 1
        qA = 2 * hx + hy
        qB = 2 * hy + hx

        barrier_sem = pltpu.get_barrier_semaphore()
        for nbr in [xp, yp]:
            pl.semaphore_signal(
                barrier_sem, inc=1,
                device_id=(nbr,), device_id_type=pl.DeviceIdType.MESH,
            )
        pl.semaphore_wait(barrier_sem, 2)

        def rdma(src, dst, sem, dev):
            return pltpu.make_async_remote_copy(
                src_ref=src, dst_ref=dst,
                send_sem=send_sems.at[sem], recv_sem=recv_sems.at[sem],
                device_id=(dev,), device_id_type=pl.DeviceIdType.MESH,
            )

        sA0[pl.ds((1 - hy) * q, q), :] = x_ref[
            pl.ds((1 - hx) * h + (1 - hy) * q, q), :].astype(bf16)
        sB0[pl.ds((1 - hx) * q, q), :] = x_ref[
            pl.ds(b_off + (1 - hy) * h + (1 - hx) * q, q), :].astype(bf16)

        a0a = rdma(sA0.at[pl.ds((1 - hy) * q, q)],
                   rA1.at[pl.ds((1 - hy) * q, q)], 0, xp)
        b0a = rdma(sB0.at[pl.ds((1 - hx) * q, q)],
                   rB1.at[pl.ds((1 - hx) * q, q)], 1, yp)
        a0a.start()
        b0a.start()

        sA0[pl.ds(hy * q, q), :] = x_ref[
            pl.ds((1 - hx) * h + hy * q, q), :].astype(bf16)
        sB0[pl.ds(hx * q, q), :] = x_ref[
            pl.ds(b_off + (1 - hy) * h + hx * q, q), :].astype(bf16)
        a0b = rdma(sA0.at[pl.ds(hy * q, q)],
                   rA1.at[pl.ds(hy * q, q)], 2, xp)
        b0b = rdma(sB0.at[pl.ds(hx * q, q)],
                   rB1.at[pl.ds(hx * q, q)], 3, yp)
        a0b.start()
        b0b.start()

        a0a.wait_recv()
        fwdA[:, :] = (x_ref[pl.ds(hx * h + (1 - hy) * q, q), :]
                      + rA1[pl.ds((1 - hy) * q, q), :].astype(f32)
                      ).astype(bf16)
        a1 = rdma(fwdA, rA2, 4, yp)
        a1.start()

        b0a.wait_recv()
        fwdB[:, :] = (x_ref[pl.ds(b_off + hy * h + (1 - hx) * q, q), :]
                      + rB1[pl.ds((1 - hx) * q, q), :].astype(f32)
                      ).astype(bf16)
        b1 = rdma(fwdB, rB2, 5, xp)
        b1.start()

        a0b.wait_recv()
        a1.wait_recv()
        sA = (x_ref[pl.ds(hx * h + hy * q, q), :]
              + rA1[pl.ds(hy * q, q), :].astype(f32)
              + rA2[:, :].astype(f32))
        rA = jnp.maximum(sA, 0.0)
        fA = jnp.tanh(sA) * sA * sA + rA * rA * rA
        out_ref[pl.ds(qA * q, q), :] = fA
        obuf[pl.ds(qA * q, q), :] = fA.astype(bf16)
        a2 = rdma(obuf.at[pl.ds(qA * q, q)],
                  obuf.at[pl.ds(qA * q, q)], 6, yp)
        a3a = rdma(obuf.at[pl.ds(qA * q, q)],
                   obuf.at[pl.ds(qA * q, q)], 7, xp)
        a2.start()
        a3a.start()

        b0b.wait_recv()
        b1.wait_recv()
        sB = (x_ref[pl.ds(b_off + hy * h + hx * q, q), :]
              + rB1[pl.ds(hx * q, q), :].astype(f32)
              + rB2[:, :].astype(f32))
        rB = jnp.maximum(sB, 0.0)
        fB = jnp.tanh(sB) * sB * sB + rB * rB * rB
        out_ref[pl.ds(b_off + qB * q, q), :] = fB
        obuf[pl.ds(b_off + qB * q, q), :] = fB.astype(bf16)
        b2 = rdma(obuf.at[pl.ds(b_off + qB * q, q)],
                  obuf.at[pl.ds(b_off + qB * q, q)], 8, xp)
        b3a = rdma(obuf.at[pl.ds(b_off + qB * q, q)],
                   obuf.at[pl.ds(b_off + qB * q, q)], 9, yp)
        b2.start()
        b3a.start()

        qA2 = 2 * hx + (1 - hy)
        qB2 = 2 * hy + (1 - hx)
        a2.wait_recv()
        a3b = rdma(obuf.at[pl.ds(qA2 * q, q)],
                   obuf.at[pl.ds(qA2 * q, q)], 10, xp)
        a3b.start()
        b2.wait_recv()
        b3b = rdma(obuf.at[pl.ds(b_off + qB2 * q, q)],
                   obuf.at[pl.ds(b_off + qB2 * q, q)], 11, yp)
        b3b.start()

        out_ref[pl.ds(qA2 * q, q), :] = obuf[pl.ds(qA2 * q, q), :].astype(f32)
        out_ref[pl.ds(b_off + qB2 * q, q), :] = obuf[
            pl.ds(b_off + qB2 * q, q), :].astype(f32)

        qA3a = 2 * (1 - hx) + hy
        qA3b = 2 * (1 - hx) + (1 - hy)
        qB3a = 2 * (1 - hy) + hx
        qB3b = 2 * (1 - hy) + (1 - hx)
        a3a.wait_recv()
        out_ref[pl.ds(qA3a * q, q), :] = obuf[
            pl.ds(qA3a * q, q), :].astype(f32)
        b3a.wait_recv()
        out_ref[pl.ds(b_off + qB3a * q, q), :] = obuf[
            pl.ds(b_off + qB3a * q, q), :].astype(f32)
        a3b.wait_recv()
        out_ref[pl.ds(qA3b * q, q), :] = obuf[
            pl.ds(qA3b * q, q), :].astype(f32)
        b3b.wait_recv()
        out_ref[pl.ds(b_off + qB3b * q, q), :] = obuf[
            pl.ds(b_off + qB3b * q, q), :].astype(f32)

        for d in [a0a, b0a, a0b, b0b, a1, b1, a2, a3a, b2, b3a, a3b, b3b]:
            d.wait_send()

    return pl.pallas_call(
        body,
        out_shape=jax.ShapeDtypeStruct((m, n), jnp.float32),
        in_specs=[pl.BlockSpec(memory_space=pltpu.VMEM)],
        out_specs=pl.BlockSpec(memory_space=pltpu.VMEM),
        scratch_shapes=[
            pltpu.VMEM((h, n), bf16),
            pltpu.VMEM((h, n), bf16),
            pltpu.VMEM((h, n), bf16),
            pltpu.VMEM((h, n), bf16),
            pltpu.VMEM((q, n), bf16),
            pltpu.VMEM((q, n), bf16),
            pltpu.VMEM((q, n), bf16),
            pltpu.VMEM((q, n), bf16),
            pltpu.VMEM((m, n), bf16),
            pltpu.SemaphoreType.DMA((12,)),
            pltpu.SemaphoreType.DMA((12,)),
        ],
        compiler_params=pltpu.CompilerParams(collective_id=0),
    )(t)


# device time: 18521 ns/iter; 4.3145x vs baseline; 1.0685x over previous
import jax
import jax.numpy as jnp
from jax import lax
from jax.experimental import pallas as pl
from jax.experimental.pallas import tpu as pltpu

N_DEV = 4


def kernel(t):
    m, n = t.shape
    h = m // 4
    q = m // 8
    c = m // 16
    b_off = m // 2
    bf16 = jnp.bfloat16
    f32 = jnp.float32

    def body(x_ref, out_ref, sA0, sB0, rA1, rB1, rA2, rB2, fwdA, fwdB,
             obuf, send_sems, recv_sems):
        my = lax.axis_index("i")
        hx = my // 2
        hy = jnp.bitwise_xor(my, my // 2) % 2
        xp = 3 - my
        yp = my ^ 1
        qA = 2 * hx + hy
        qB = 2 * hy + hx

        barrier_sem = pltpu.get_barrier_semaphore()
        for nbr in [xp, yp]:
            pl.semaphore_signal(
                barrier_sem, inc=1,
                device_id=(nbr,), device_id_type=pl.DeviceIdType.MESH,
            )
        pl.semaphore_wait(barrier_sem, 2)

        sem_ctr = [0]
        all_descs = []

        def rdma(src, dst, dev):
            i = sem_ctr[0]
            sem_ctr[0] += 1
            d = pltpu.make_async_remote_copy(
                src_ref=src, dst_ref=dst,
                send_sem=send_sems.at[i], recv_sem=recv_sems.at[i],
                device_id=(dev,), device_id_type=pl.DeviceIdType.MESH,
            )
            all_descs.append(d)
            return d

        pA = dict(off=0, p1=xp, p2=yp, k1=hx, k2=hy, sbuf=sA0, r1=rA1,
                  r2=rA2, fwd=fwdA)
        pB = dict(off=b_off, p1=yp, p2=xp, k1=hy, k2=hx, sbuf=sB0, r1=rB1,
                  r2=rB2, fwd=fwdB)

        st0 = {id(pA): [], id(pB): []}
        for sub in range(4):
            loc = sub // 2
            k = sub % 2
            for P in (pA, pB):
                lo = (1 - P["k2"]) * q + k * c if loc == 0 else (
                    P["k2"] * q + k * c)
                src_rows = P["off"] + (1 - P["k1"]) * h + lo
                P["sbuf"][pl.ds(lo, c), :] = x_ref[
                    pl.ds(src_rows, c), :].astype(bf16)
                d = rdma(P["sbuf"].at[pl.ds(lo, c)],
                         P["r1"].at[pl.ds(lo, c)], P["p1"])
                d.start()
                st0[id(P)].append(d)

        st1 = {id(pA): [], id(pB): []}
        for k in range(2):
            for P in (pA, pB):
                lo = (1 - P["k2"]) * q + k * c
                st0[id(P)][k].wait_recv()
                P["fwd"][pl.ds(k * c, c), :] = (
                    x_ref[pl.ds(P["off"] + P["k1"] * h + lo, c), :]
                    + P["r1"][pl.ds(lo, c), :].astype(f32)).astype(bf16)
                d = rdma(P["fwd"].at[pl.ds(k * c, c)],
                         P["r2"].at[pl.ds(k * c, c)], P["p2"])
                d.start()
                st1[id(P)].append(d)

        ag2 = {id(pA): [], id(pB): []}
        ag3a = {id(pA): [], id(pB): []}
        for k in range(2):
            for P in (pA, pB):
                myq = 2 * P["k1"] + P["k2"]
                lo = P["k2"] * q + k * c
                st0[id(P)][2 + k].wait_recv()
                st1[id(P)][k].wait_recv()
                s = (x_ref[pl.ds(P["off"] + P["k1"] * h + lo, c), :]
                     + P["r1"][pl.ds(lo, c), :].astype(f32)
                     + P["r2"][pl.ds(k * c, c), :].astype(f32))
                r = jnp.maximum(s, 0.0)
                fv = jnp.tanh(s) * s * s + r * r * r
                orow = P["off"] + myq * q + k * c
                out_ref[pl.ds(orow, c), :] = fv
                obuf[pl.ds(orow, c), :] = fv.astype(bf16)
                d2 = rdma(obuf.at[pl.ds(orow, c)],
                          obuf.at[pl.ds(orow, c)], P["p2"])
                d3 = rdma(obuf.at[pl.ds(orow, c)],
                          obuf.at[pl.ds(orow, c)], P["p1"])
                d2.start()
                d3.start()
                ag2[id(P)].append(d2)
                ag3a[id(P)].append(d3)

        ag3b = {id(pA): [], id(pB): []}
        for k in range(2):
            for P in (pA, pB):
                q2 = 2 * P["k1"] + (1 - P["k2"])
                orow = P["off"] + q2 * q + k * c
                ag2[id(P)][k].wait_recv()
                d = rdma(obuf.at[pl.ds(orow, c)],
                         obuf.at[pl.ds(orow, c)], P["p1"])
                d.start()
                ag3b[id(P)].append(d)
                out_ref[pl.ds(orow, c), :] = obuf[
                    pl.ds(orow, c), :].astype(f32)

        for k in range(2):
            for P in (pA, pB):
                q3a = 2 * (1 - P["k1"]) + P["k2"]
                orow = P["off"] + q3a * q + k * c
                ag3a[id(P)][k].wait_recv()
                out_ref[pl.ds(orow, c), :] = obuf[
                    pl.ds(orow, c), :].astype(f32)
        for k in range(2):
            for P in (pA, pB):
                q3b = 2 * (1 - P["k1"]) + (1 - P["k2"])
                orow = P["off"] + q3b * q + k * c
                ag3b[id(P)][k].wait_recv()
                out_ref[pl.ds(orow, c), :] = obuf[
                    pl.ds(orow, c), :].astype(f32)

        for d in all_descs:
            d.wait_send()

    n_sems = 24
    return pl.pallas_call(
        body,
        out_shape=jax.ShapeDtypeStruct((m, n), jnp.float32),
        in_specs=[pl.BlockSpec(memory_space=pltpu.VMEM)],
        out_specs=pl.BlockSpec(memory_space=pltpu.VMEM),
        scratch_shapes=[
            pltpu.VMEM((h, n), bf16),
            pltpu.VMEM((h, n), bf16),
            pltpu.VMEM((h, n), bf16),
            pltpu.VMEM((h, n), bf16),
            pltpu.VMEM((q, n), bf16),
            pltpu.VMEM((q, n), bf16),
            pltpu.VMEM((q, n), bf16),
            pltpu.VMEM((q, n), bf16),
            pltpu.VMEM((m, n), bf16),
            pltpu.SemaphoreType.DMA((n_sems,)),
            pltpu.SemaphoreType.DMA((n_sems,)),
        ],
        compiler_params=pltpu.CompilerParams(collective_id=0),
    )(t)


# device time: 18246 ns/iter; 4.3795x vs baseline; 1.0151x over previous
import jax
import jax.numpy as jnp
from jax import lax
from jax.experimental import pallas as pl
from jax.experimental.pallas import tpu as pltpu

N_DEV = 4


def kernel(t):
    m, n = t.shape
    h = m // 4
    q = m // 8
    c = m // 32
    spq = q // c
    b_off = m // 2
    bf16 = jnp.bfloat16
    f32 = jnp.float32

    def body(x_ref, out_ref, sA0, sB0, rA1, rB1, rA2, rB2, fwdA, fwdB,
             obuf, send_sems, recv_sems):
        my = lax.axis_index("i")
        hx = my // 2
        hy = jnp.bitwise_xor(my, my // 2) % 2
        xp = 3 - my
        yp = my ^ 1
        qA = 2 * hx + hy
        qB = 2 * hy + hx

        barrier_sem = pltpu.get_barrier_semaphore()
        for nbr in [xp, yp]:
            pl.semaphore_signal(
                barrier_sem, inc=1,
                device_id=(nbr,), device_id_type=pl.DeviceIdType.MESH,
            )
        pl.semaphore_wait(barrier_sem, 2)

        sem_ctr = [0]
        all_descs = []

        def rdma(src, dst, dev):
            i = sem_ctr[0]
            sem_ctr[0] += 1
            d = pltpu.make_async_remote_copy(
                src_ref=src, dst_ref=dst,
                send_sem=send_sems.at[i], recv_sem=recv_sems.at[i],
                device_id=(dev,), device_id_type=pl.DeviceIdType.MESH,
            )
            all_descs.append(d)
            return d

        pA = dict(off=0, p1=xp, p2=yp, k1=hx, k2=hy, sbuf=sA0, r1=rA1,
                  r2=rA2, fwd=fwdA)
        pB = dict(off=b_off, p1=yp, p2=xp, k1=hy, k2=hx, sbuf=sB0, r1=rB1,
                  r2=rB2, fwd=fwdB)

        st0 = {id(pA): [], id(pB): []}
        for sub in range(2 * spq):
            loc = sub // spq
            k = sub % spq
            for P in (pA, pB):
                lo = (1 - P["k2"]) * q + k * c if loc == 0 else (
                    P["k2"] * q + k * c)
                src_rows = P["off"] + (1 - P["k1"]) * h + lo
                P["sbuf"][pl.ds(lo, c), :] = x_ref[
                    pl.ds(src_rows, c), :].astype(bf16)
                d = rdma(P["sbuf"].at[pl.ds(lo, c)],
                         P["r1"].at[pl.ds(lo, c)], P["p1"])
                d.start()
                st0[id(P)].append(d)

        st1 = {id(pA): [], id(pB): []}
        for k in range(spq):
            for P in (pA, pB):
                lo = (1 - P["k2"]) * q + k * c
                st0[id(P)][k].wait_recv()
                P["fwd"][pl.ds(k * c, c), :] = (
                    x_ref[pl.ds(P["off"] + P["k1"] * h + lo, c), :]
                    + P["r1"][pl.ds(lo, c), :].astype(f32)).astype(bf16)
                d = rdma(P["fwd"].at[pl.ds(k * c, c)],
                         P["r2"].at[pl.ds(k * c, c)], P["p2"])
                d.start()
                st1[id(P)].append(d)

        ag2 = {id(pA): [], id(pB): []}
        ag3a = {id(pA): [], id(pB): []}
        for k in range(spq):
            for P in (pA, pB):
                myq = 2 * P["k1"] + P["k2"]
                lo = P["k2"] * q + k * c
                st0[id(P)][spq + k].wait_recv()
                st1[id(P)][k].wait_recv()
                s = (x_ref[pl.ds(P["off"] + P["k1"] * h + lo, c), :]
                     + P["r1"][pl.ds(lo, c), :].astype(f32)
                     + P["r2"][pl.ds(k * c, c), :].astype(f32))
                r = jnp.maximum(s, 0.0)
                fv = jnp.tanh(s) * s * s + r * r * r
                orow = P["off"] + myq * q + k * c
                out_ref[pl.ds(orow, c), :] = fv
                obuf[pl.ds(orow, c), :] = fv.astype(bf16)
                d2 = rdma(obuf.at[pl.ds(orow, c)],
                          obuf.at[pl.ds(orow, c)], P["p2"])
                d3 = rdma(obuf.at[pl.ds(orow, c)],
                          obuf.at[pl.ds(orow, c)], P["p1"])
                d2.start()
                d3.start()
                ag2[id(P)].append(d2)
                ag3a[id(P)].append(d3)

        ag3b = {id(pA): [], id(pB): []}
        for k in range(spq):
            for P in (pA, pB):
                q2 = 2 * P["k1"] + (1 - P["k2"])
                orow = P["off"] + q2 * q + k * c
                ag2[id(P)][k].wait_recv()
                d = rdma(obuf.at[pl.ds(orow, c)],
                         obuf.at[pl.ds(orow, c)], P["p1"])
                d.start()
                ag3b[id(P)].append(d)
                out_ref[pl.ds(orow, c), :] = obuf[
                    pl.ds(orow, c), :].astype(f32)

        for k in range(spq):
            for P in (pA, pB):
                q3a = 2 * (1 - P["k1"]) + P["k2"]
                orow = P["off"] + q3a * q + k * c
                ag3a[id(P)][k].wait_recv()
                out_ref[pl.ds(orow, c), :] = obuf[
                    pl.ds(orow, c), :].astype(f32)
        for k in range(spq):
            for P in (pA, pB):
                q3b = 2 * (1 - P["k1"]) + (1 - P["k2"])
                orow = P["off"] + q3b * q + k * c
                ag3b[id(P)][k].wait_recv()
                out_ref[pl.ds(orow, c), :] = obuf[
                    pl.ds(orow, c), :].astype(f32)

        for d in all_descs:
            d.wait_send()

    n_sems = 12 * (q // c)
    return pl.pallas_call(
        body,
        out_shape=jax.ShapeDtypeStruct((m, n), jnp.float32),
        in_specs=[pl.BlockSpec(memory_space=pltpu.VMEM)],
        out_specs=pl.BlockSpec(memory_space=pltpu.VMEM),
        scratch_shapes=[
            pltpu.VMEM((h, n), bf16),
            pltpu.VMEM((h, n), bf16),
            pltpu.VMEM((h, n), bf16),
            pltpu.VMEM((h, n), bf16),
            pltpu.VMEM((q, n), bf16),
            pltpu.VMEM((q, n), bf16),
            pltpu.VMEM((q, n), bf16),
            pltpu.VMEM((q, n), bf16),
            pltpu.VMEM((m, n), bf16),
            pltpu.SemaphoreType.DMA((n_sems,)),
            pltpu.SemaphoreType.DMA((n_sems,)),
        ],
        compiler_params=pltpu.CompilerParams(collective_id=0),
    )(t)
